# baseline (device time: 118222 ns/iter reference)
import jax
import jax.numpy as jnp
from jax import lax
from jax.experimental import pallas as pl
from jax.experimental.pallas import tpu as pltpu

M = 1536
N = 1536
K = 768
HALF = M // 2


def kernel(A, B):
    def body(a_ref, b_ref, out_ref, p_ref, comm_ref,
             send_sem_x, recv_sem_x, send_sem_y, recv_sem_y):
        my_x = lax.axis_index("x")
        my_y = lax.axis_index("y")
        peer_x = (1 - my_x, my_y)
        peer_y = (my_x, 1 - my_y)

        barrier = pltpu.get_barrier_semaphore()
        for nbr in (peer_x, peer_y):
            pl.semaphore_signal(barrier, inc=1, device_id=nbr,
                                device_id_type=pl.DeviceIdType.MESH)
        pl.semaphore_wait(barrier, 2)

        row0 = my_y * HALF

        p_ref[...] = jnp.dot(a_ref[pl.ds(row0, HALF), :], b_ref[...],
                             preferred_element_type=jnp.float32)

        rdma_x = pltpu.make_async_remote_copy(
            src_ref=p_ref, dst_ref=comm_ref,
            send_sem=send_sem_x, recv_sem=recv_sem_x,
            device_id=peer_x, device_id_type=pl.DeviceIdType.MESH)
        rdma_x.start()
        rdma_x.wait()

        out_ref[pl.ds(row0, HALF), :] = p_ref[...] + comm_ref[...]

        rdma_y = pltpu.make_async_remote_copy(
            src_ref=out_ref.at[pl.ds(row0, HALF), :],
            dst_ref=out_ref.at[pl.ds(row0, HALF), :],
            send_sem=send_sem_y, recv_sem=recv_sem_y,
            device_id=peer_y, device_id_type=pl.DeviceIdType.MESH)
        rdma_y.start()
        rdma_y.wait()

    return pl.pallas_call(
        body,
        out_shape=jax.ShapeDtypeStruct((M, N), jnp.float32),
        in_specs=[pl.BlockSpec(memory_space=pltpu.VMEM)] * 2,
        out_specs=pl.BlockSpec(memory_space=pltpu.VMEM),
        scratch_shapes=[
            pltpu.VMEM((HALF, N), jnp.float32),
            pltpu.VMEM((HALF, N), jnp.float32),
            pltpu.SemaphoreType.DMA,
            pltpu.SemaphoreType.DMA,
            pltpu.SemaphoreType.DMA,
            pltpu.SemaphoreType.DMA,
        ],
        compiler_params=pltpu.CompilerParams(collective_id=0),
    )(A, B)


# device time: 73903 ns/iter; 1.5997x vs baseline; 1.5997x over previous
import jax
import jax.numpy as jnp
from jax import lax
from jax.experimental import pallas as pl
from jax.experimental.pallas import tpu as pltpu

M = 1536
N = 1536
K = 768
HALF = M // 2
C = 12
W = N // C


def kernel(A, B):
    def body(a_ref, b_ref, out_ref, p_ref, comm_ref,
             send_x, recv_x, send_y, recv_y):
        my_x = lax.axis_index("x")
        my_y = lax.axis_index("y")
        peer_x = (1 - my_x, my_y)
        peer_y = (my_x, 1 - my_y)

        barrier = pltpu.get_barrier_semaphore()
        for nbr in (peer_x, peer_y):
            pl.semaphore_signal(barrier, inc=1, device_id=nbr,
                                device_id_type=pl.DeviceIdType.MESH)
        pl.semaphore_wait(barrier, 2)

        row0 = my_y * HALF
        a_half = a_ref[pl.ds(row0, HALF), :]

        def rdma_x_c(c):
            return pltpu.make_async_remote_copy(
                src_ref=p_ref.at[:, pl.ds(c * W, W)],
                dst_ref=comm_ref.at[:, pl.ds(c * W, W)],
                send_sem=send_x.at[c], recv_sem=recv_x.at[c],
                device_id=peer_x, device_id_type=pl.DeviceIdType.MESH)

        def rdma_y_c(c):
            return pltpu.make_async_remote_copy(
                src_ref=out_ref.at[pl.ds(row0, HALF), pl.ds(c * W, W)],
                dst_ref=out_ref.at[pl.ds(row0, HALF), pl.ds(c * W, W)],
                send_sem=send_y.at[c], recv_sem=recv_y.at[c],
                device_id=peer_y, device_id_type=pl.DeviceIdType.MESH)

        for c in range(C):
            p_ref[:, pl.ds(c * W, W)] = jnp.dot(
                a_half, b_ref[:, pl.ds(c * W, W)],
                preferred_element_type=jnp.float32)
            rdma_x_c(c).start()

        for c in range(C):
            r = rdma_x_c(c)
            r.wait_recv()
            r.wait_send()
            out_ref[pl.ds(row0, HALF), pl.ds(c * W, W)] = (
                p_ref[:, pl.ds(c * W, W)] + comm_ref[:, pl.ds(c * W, W)])
            rdma_y_c(c).start()

        for c in range(C):
            r = rdma_y_c(c)
            r.wait_recv()
            r.wait_send()

    return pl.pallas_call(
        body,
        out_shape=jax.ShapeDtypeStruct((M, N), jnp.float32),
        in_specs=[pl.BlockSpec(memory_space=pltpu.VMEM)] * 2,
        out_specs=pl.BlockSpec(memory_space=pltpu.VMEM),
        scratch_shapes=[
            pltpu.VMEM((HALF, N), jnp.float32),
            pltpu.VMEM((HALF, N), jnp.float32),
            pltpu.SemaphoreType.DMA((C,)),
            pltpu.SemaphoreType.DMA((C,)),
            pltpu.SemaphoreType.DMA((C,)),
            pltpu.SemaphoreType.DMA((C,)),
        ],
        compiler_params=pltpu.CompilerParams(collective_id=0),
    )(A, B)


# device time: 70420 ns/iter; 1.6788x vs baseline; 1.0495x over previous
import jax
import jax.numpy as jnp
from jax import lax
from jax.experimental import pallas as pl
from jax.experimental.pallas import tpu as pltpu

M = 1536
N = 1536
K = 768
HALF = M // 2
C = 12
W = N // C


def kernel(A, B):
    def body(a_ref, b_ref, out_ref, p_ref, comm_ref,
             send_x, recv_x, send_y, recv_y):
        my_x = lax.axis_index("x")
        my_y = lax.axis_index("y")
        peer_x = (1 - my_x, my_y)
        peer_y = (my_x, 1 - my_y)

        barrier = pltpu.get_barrier_semaphore()
        for nbr in (peer_x, peer_y):
            pl.semaphore_signal(barrier, inc=1, device_id=nbr,
                                device_id_type=pl.DeviceIdType.MESH)
        pl.semaphore_wait(barrier, 2)

        row0 = my_y * HALF
        a_half = a_ref[pl.ds(row0, HALF), :]

        def rdma_x_c(c):
            return pltpu.make_async_remote_copy(
                src_ref=p_ref.at[:, pl.ds(c * W, W)],
                dst_ref=comm_ref.at[:, pl.ds(c * W, W)],
                send_sem=send_x.at[c], recv_sem=recv_x.at[c],
                device_id=peer_x, device_id_type=pl.DeviceIdType.MESH)

        def rdma_y_c(c):
            return pltpu.make_async_remote_copy(
                src_ref=out_ref.at[pl.ds(row0, HALF), pl.ds(c * W, W)],
                dst_ref=out_ref.at[pl.ds(row0, HALF), pl.ds(c * W, W)],
                send_sem=send_y.at[c], recv_sem=recv_y.at[c],
                device_id=peer_y, device_id_type=pl.DeviceIdType.MESH)

        def compute_and_send(c):
            p_ref[:, pl.ds(c * W, W)] = jnp.dot(
                a_half, b_ref[:, pl.ds(c * W, W)],
                preferred_element_type=jnp.float32)
            rdma_x_c(c).start()

        compute_and_send(0)
        compute_and_send(1)
        for c in range(C):
            if c + 2 < C:
                compute_and_send(c + 2)
            r = rdma_x_c(c)
            r.wait_recv()
            r.wait_send()
            out_ref[pl.ds(row0, HALF), pl.ds(c * W, W)] = (
                p_ref[:, pl.ds(c * W, W)] + comm_ref[:, pl.ds(c * W, W)])
            rdma_y_c(c).start()

        for c in range(C):
            r = rdma_y_c(c)
            r.wait_recv()
            r.wait_send()

    return pl.pallas_call(
        body,
        out_shape=jax.ShapeDtypeStruct((M, N), jnp.float32),
        in_specs=[pl.BlockSpec(memory_space=pltpu.VMEM)] * 2,
        out_specs=pl.BlockSpec(memory_space=pltpu.VMEM),
        scratch_shapes=[
            pltpu.VMEM((HALF, N), jnp.float32),
            pltpu.VMEM((HALF, N), jnp.float32),
            pltpu.SemaphoreType.DMA((C,)),
            pltpu.SemaphoreType.DMA((C,)),
            pltpu.SemaphoreType.DMA((C,)),
            pltpu.SemaphoreType.DMA((C,)),
        ],
        compiler_params=pltpu.CompilerParams(collective_id=0),
    )(A, B)
